# baseline (speedup 1.0000x reference)
import functools

import jax
import jax.numpy as jnp
from jax import lax
from jax.experimental import pallas as pl
from jax.experimental.pallas import tpu as pltpu
from jax.experimental.pallas import tpu_sc as plsc

_NC, _NS = 2, 16
_NW = _NC * _NS
_NBUF = 4


@functools.lru_cache(maxsize=None)
def _make_gather(nrows, dim):
    rows_w = nrows // _NW
    n = rows_w // 128
    half = dim
    mesh = plsc.VectorSubcoreMesh(core_axis_name="c", subcore_axis_name="s")

    @functools.partial(
        pl.kernel,
        out_type=jax.ShapeDtypeStruct((nrows // 2, 2 * dim), jnp.float32),
        mesh=mesh,
        scratch_types=[
            pltpu.VMEM((n, 128), jnp.int32),
            [pltpu.VMEM((128,), jnp.int32) for _ in range(_NBUF)],
            [pltpu.VMEM((128, 2 * dim), jnp.float32) for _ in range(_NBUF)],
            [pltpu.VMEM((64, 2 * dim), jnp.float32) for _ in range(_NBUF)],
            [pltpu.SemaphoreType.DMA for _ in range(_NBUF)],
            [pltpu.SemaphoreType.DMA for _ in range(_NBUF)],
        ],
        compiler_params=pltpu.CompilerParams(
            use_tc_tiling_on_sc=True, needs_layout_passes=False
        ),
    )
    def gather(idx_hbm, table_hbm, out_hbm, idx_v, pidx, bufp, bufo, sg, sw):
        wid = lax.axis_index("s") * _NC + lax.axis_index("c")
        i0 = pl.multiple_of(wid * n, 8)
        pltpu.sync_copy(idx_hbm.at[pl.ds(i0, n)], idx_v)

        iota = lax.iota(jnp.int32, 16)

        def prep_fire_g(k, b):
            @pl.loop(0, 8)
            def _g(g):
                v = idx_v[k, pl.ds(pl.multiple_of(g * 16, 16), 16)]
                pidx[b][pl.ds(pl.multiple_of(g * 16, 16), 16)] = v >> 1

            pltpu.async_copy(table_hbm.at[pidx[b]], bufp[b], sg[b])

        def drain_g(b):
            pltpu.make_async_copy(
                table_hbm.at[pidx[b]], bufp[b], sg[b]
            ).wait()

        def select(k, b):
            @pl.loop(0, 2)
            def _h(h):
                kvec = jnp.full((16,), k, jnp.int32)
                hvec = jnp.full((16,), h * half, jnp.int32)

                @pl.loop(0, 4)
                def _g(g):
                    tvec = iota + g * 16
                    rows = 2 * tvec + h
                    idxv = plsc.load_gather(idx_v, [kvec, rows])
                    off = (idxv & 1) * half
                    for c in range(half):
                        vals = plsc.load_gather(bufp[b], [rows, off + c])
                        plsc.store_scatter(bufo[b], [tvec, hvec + c], vals)

        def fire_w(k, b):
            r0 = pl.multiple_of(wid * (rows_w // 2) + k * 64, 8)
            pltpu.async_copy(bufo[b], out_hbm.at[pl.ds(r0, 64)], sw[b])

        def wait_w(b):
            pltpu.make_async_copy(
                bufo[b], out_hbm.at[pl.ds(0, 64)], sw[b]
            ).wait()

        for b in range(_NBUF - 1):
            prep_fire_g(b, b)

        @pl.loop(0, n // _NBUF)
        def _m(m):
            for s in range(_NBUF):
                k = m * _NBUF + s
                drain_g(s)
                select(k, s)
                fire_w(k, s)
                bn = (s + _NBUF - 1) % _NBUF

                @pl.when(jnp.logical_and(k >= 1, k < n - (_NBUF - 1)))
                def _wait():
                    wait_w(bn)

                @pl.when(k < n - (_NBUF - 1))
                def _fire():
                    prep_fire_g(k + _NBUF - 1, bn)

        for b in range(_NBUF):
            wait_w(b)

    return gather


def kernel(indices, weight):
    batch, hist = indices.shape
    dim = weight.shape[1]
    nrows = batch * hist
    out = _make_gather(nrows, dim)(
        indices.reshape(-1, 128), weight.reshape(-1, 2 * dim)
    )
    return out.reshape(batch, hist, dim)


# speedup vs baseline: 2.3841x; 2.3841x over previous
import functools

import jax
import jax.numpy as jnp
from jax import lax
from jax.experimental import pallas as pl
from jax.experimental.pallas import tpu as pltpu
from jax.experimental.pallas import tpu_sc as plsc

_NC, _NS = 2, 16
_NW = _NC * _NS
_CHUNK = 256
_NBUF = 4


@functools.lru_cache(maxsize=None)
def _make_gather(nrows, dim):
    rows_w = nrows // _NW
    n = rows_w // _CHUNK
    nidx = rows_w // 128
    s_per_c = _CHUNK // 128
    mesh = plsc.VectorSubcoreMesh(core_axis_name="c", subcore_axis_name="s")

    @functools.partial(
        pl.kernel,
        out_type=jax.ShapeDtypeStruct((nrows, dim), jnp.float32),
        mesh=mesh,
        scratch_types=[
            pltpu.VMEM((nidx, 128), jnp.int32),
            [pltpu.VMEM((_CHUNK, dim), jnp.float32) for _ in range(_NBUF)],
            [pltpu.SemaphoreType.DMA for _ in range(_NBUF)],
            [pltpu.SemaphoreType.DMA for _ in range(_NBUF)],
        ],
        compiler_params=pltpu.CompilerParams(
            use_tc_tiling_on_sc=False, needs_layout_passes=True
        ),
    )
    def gather(idx_hbm, table_hbm, out_hbm, idx_v, buf, sg, sw):
        wid = lax.axis_index("s") * _NC + lax.axis_index("c")
        i0 = pl.multiple_of(wid * nidx, 8)
        pltpu.sync_copy(idx_hbm.at[pl.ds(i0, nidx)], idx_v)

        def fire_g(k, b):
            for i in range(s_per_c):
                pltpu.async_copy(
                    table_hbm.at[idx_v.at[k * s_per_c + i]],
                    buf[b].at[pl.ds(i * 128, 128)],
                    sg[b],
                )

        def drain_g(b):
            for _ in range(s_per_c):
                pltpu.make_async_copy(
                    table_hbm.at[idx_v.at[0]], buf[b].at[pl.ds(0, 128)], sg[b]
                ).wait()

        def fire_w(k, b):
            r0 = pl.multiple_of(wid * rows_w + k * _CHUNK, 8)
            pltpu.async_copy(buf[b], out_hbm.at[pl.ds(r0, _CHUNK)], sw[b])

        def wait_w(b):
            pltpu.make_async_copy(
                buf[b], out_hbm.at[pl.ds(0, _CHUNK)], sw[b]
            ).wait()

        for b in range(_NBUF - 1):
            fire_g(b, b)

        @pl.loop(0, n // _NBUF)
        def _m(m):
            for s in range(_NBUF):
                k = m * _NBUF + s
                drain_g(s)
                fire_w(k, s)
                bn = (s + _NBUF - 1) % _NBUF

                @pl.when(jnp.logical_and(k >= 1, k < n - (_NBUF - 1)))
                def _wait():
                    wait_w(bn)

                @pl.when(k < n - (_NBUF - 1))
                def _fire():
                    fire_g(k + _NBUF - 1, bn)

        for b in range(_NBUF):
            wait_w(b)

    return gather


def kernel(indices, weight):
    batch, hist = indices.shape
    dim = weight.shape[1]
    nrows = batch * hist
    out = _make_gather(nrows, dim)(indices.reshape(-1, 128), weight)
    return out.reshape(batch, hist, dim)


# speedup vs baseline: 2.5087x; 1.0523x over previous
import functools

import jax
import jax.numpy as jnp
from jax import lax
from jax.experimental import pallas as pl
from jax.experimental.pallas import tpu as pltpu
from jax.experimental.pallas import tpu_sc as plsc

_NC, _NS = 2, 16
_NW = _NC * _NS
_CHUNK = 256
_NBUF = 4


@functools.lru_cache(maxsize=None)
def _make_gather(nrows, dim):
    rows_w = nrows // _NW
    n = rows_w // _CHUNK
    nidx = rows_w // 128
    s_per_c = _CHUNK // 128
    mesh = plsc.VectorSubcoreMesh(core_axis_name="c", subcore_axis_name="s")

    @functools.partial(
        pl.kernel,
        out_type=jax.ShapeDtypeStruct((nrows, dim), jnp.float32),
        mesh=mesh,
        scratch_types=[
            pltpu.VMEM((nidx, 128), jnp.int32),
            [pltpu.VMEM((_CHUNK, dim), jnp.float32) for _ in range(_NBUF)],
            [pltpu.SemaphoreType.DMA for _ in range(_NBUF)],
            [pltpu.SemaphoreType.DMA for _ in range(_NBUF)],
        ],
        compiler_params=pltpu.CompilerParams(
            use_tc_tiling_on_sc=False, needs_layout_passes=True
        ),
    )
    def gather(idx_hbm, table_hbm, out_hbm, idx_v, buf, sg, sw):
        wid = lax.axis_index("s") * _NC + lax.axis_index("c")
        i0 = pl.multiple_of(wid * nidx, 8)
        pltpu.sync_copy(idx_hbm.at[pl.ds(i0, nidx)], idx_v)

        def fire_g(k, b):
            for i in range(s_per_c):
                pltpu.async_copy(
                    table_hbm.at[idx_v.at[k * s_per_c + i]],
                    buf[b].at[pl.ds(i * 128, 128)],
                    sg[b],
                )

        def drain_g(b):
            for _ in range(s_per_c):
                pltpu.make_async_copy(
                    table_hbm.at[idx_v.at[0]], buf[b].at[pl.ds(0, 128)], sg[b]
                ).wait()

        def fire_w(k, b):
            r0 = pl.multiple_of(wid * rows_w + k * _CHUNK, 8)
            pltpu.async_copy(buf[b], out_hbm.at[pl.ds(r0, _CHUNK)], sw[b])

        def wait_w(b):
            pltpu.make_async_copy(
                buf[b], out_hbm.at[pl.ds(0, _CHUNK)], sw[b]
            ).wait()

        for b in range(_NBUF - 1):
            fire_g(b, b)

        @pl.loop(0, n // _NBUF)
        def _m(m):
            for s in range(_NBUF):
                k = m * _NBUF + s
                drain_g(s)
                fire_w(k, s)
                bn = (s + _NBUF - 1) % _NBUF

                @pl.when(jnp.logical_and(k >= 1, k < n - (_NBUF - 1)))
                def _wait():
                    wait_w(bn)

                @pl.when(k < n - (_NBUF - 1))
                def _fire():
                    fire_g(k + _NBUF - 1, bn)

        for b in range(_NBUF):
            wait_w(b)

    return gather


def kernel(indices, weight):
    batch, hist = indices.shape
    dim = weight.shape[1]
    nrows = batch * hist
    table = jnp.pad(weight, ((0, 0), (0, 128 - dim))).reshape(-1, dim)
    out = _make_gather(nrows, dim)(indices.reshape(-1, 128) * 2, table)
    return out.reshape(batch, hist, dim)
